# baseline (speedup 1.0000x reference)
import jax
import jax.numpy as jnp
from jax import lax
from jax.experimental import pallas as pl
from jax.experimental.pallas import tpu as pltpu
from jax.experimental.pallas import tpu_sc as plsc

_B = 4096
_F = 26
_V = 100001
_D = 16
_C = 4
_FP = 32
_RP = 2600032

_info = plsc.get_sparse_core_info()
_NC = _info.num_cores
_NS = _info.num_subcores
_L = _info.num_lanes
_NW = _NC * _NS
_BPW = _B // _NW
_IPW = _BPW * _F
_CH = 128
_NCH = _IPW // _CH
_OD = (_F + 1) * _D


def _body(idx_hbm, cont_hbm, table_hbm, wb_hbm, out_hbm,
          idx2, gidx, rows, slab, cont_v, wb_v, gsem, ssem):
  wid = lax.axis_index("s") * _NC + lax.axis_index("c")
  base_b = wid * _BPW

  pltpu.sync_copy(idx_hbm.at[pl.ds(base_b, _BPW)], idx2)
  for c in range(_C):
    pltpu.sync_copy(cont_hbm.at[pl.ds(c * _B + base_b, _BPW)],
                    cont_v.at[pl.ds(c * _BPW, _BPW)])
  pltpu.sync_copy(wb_hbm, wb_v)

  iota = lax.iota(jnp.int32, _L)

  fhi = iota + (_F - _L)
  vlo = iota * _V
  vhi = fhi * _V

  def gbody(b, _):
    bv = jnp.full((_L,), b, jnp.int32)
    lo = plsc.load_gather(idx2, [bv, iota])
    hi = plsc.load_gather(idx2, [bv, fhi])
    gidx[pl.ds(b * _F, _L)] = lo + vlo
    gidx[pl.ds(b * _F + (_F - _L), _L)] = hi + vhi
    return 0

  lax.fori_loop(0, _BPW, gbody, 0)

  gcopies = [
      pltpu.async_copy(table_hbm.at[gidx.at[pl.ds(j * _CH, _CH)]],
                       rows.at[pl.ds(j * _CH, _CH)], gsem)
      for j in range(_NCH)
  ]

  wsp = [[wb_v[c * _D + d] for d in range(_D)] for c in range(_C)]
  bsp = [wb_v[_C * _D + d] for d in range(_D)]

  def mbody(bc, _):
    bs = pl.ds(bc * _L, _L)
    xc = [cont_v[pl.ds(c * _BPW + bc * _L, _L)] for c in range(_C)]
    for d in range(_D):
      acc = bsp[d]
      for c in range(_C):
        acc = acc + xc[c] * wsp[c][d]
      slab[_F * _D + d, bs] = acc
    return 0

  lax.fori_loop(0, _BPW // _L, mbody, 0)

  for cp in gcopies:
    cp.wait()

  dcol = [jnp.full((_L,), d, jnp.int32) for d in range(_D)]

  def tbody(bc, _):
    bvec = bc * _L + iota
    bs = pl.ds(bc * _L, _L)
    for f in range(_F):
      rvec = bvec * _F + f
      for d in range(_D):
        slab[f * _D + d, bs] = plsc.load_gather(rows, [rvec, dcol[d]])
    return 0

  lax.fori_loop(0, _BPW // _L, tbody, 0)

  pltpu.async_copy(slab, out_hbm.at[:, pl.ds(base_b, _BPW)], ssem).wait()


_sc_call = pl.kernel(
    _body,
    out_type=jax.ShapeDtypeStruct((_OD, _B), jnp.float32),
    mesh=plsc.VectorSubcoreMesh(core_axis_name="c", subcore_axis_name="s"),
    compiler_params=pltpu.CompilerParams(use_tc_tiling_on_sc=False,
                                         needs_layout_passes=False),
    scratch_types=[
        pltpu.VMEM((_BPW, _FP), jnp.int32),
        pltpu.VMEM((_IPW,), jnp.int32),
        pltpu.VMEM((_IPW, _D), jnp.float32),
        pltpu.VMEM((_OD, _BPW), jnp.float32),
        pltpu.VMEM((_BPW * _C,), jnp.float32),
        pltpu.VMEM(((_C + 1) * _D, _L), jnp.float32),
        pltpu.SemaphoreType.DMA,
        pltpu.SemaphoreType.DMA,
    ],
)


def kernel(indices, cont, tables, means, variances, W, b):
  scale = 1.0 / jnp.sqrt(variances)
  w_fold = W * scale[:, None]
  b_fold = b - (means * scale) @ W
  wb_flat = lax.optimization_barrier(
      jnp.tile(jnp.concatenate([w_fold.reshape(-1), b_fold])[:, None],
               (1, _L)))
  idx_pad = lax.optimization_barrier(
      jnp.pad(indices.astype(jnp.int32), ((0, 0), (0, _FP - _F))))
  cont_flat = lax.optimization_barrier(cont.T.reshape(-1))
  tflat = jnp.pad(tables.reshape(-1), (0, (_RP - _F * _V) * _D))
  table_flat = lax.optimization_barrier(tflat).reshape(_RP, _D)
  out_t = _sc_call(idx_pad, cont_flat, table_flat, wb_flat)
  return jnp.transpose(out_t.reshape(_F + 1, _D, _B), (2, 0, 1))


# speedup vs baseline: 3.9865x; 3.9865x over previous
import jax
import jax.numpy as jnp
from jax import lax
from jax.experimental import pallas as pl
from jax.experimental.pallas import tpu as pltpu
from jax.experimental.pallas import tpu_sc as plsc

_B = 4096
_F = 26
_V = 100001
_D = 16
_C = 4
_FP = 32
_VP = 100096
_RP = _F * _VP
_VB = 4352
_NVB = _VP // _VB

_info = plsc.get_sparse_core_info()
_NC = _info.num_cores
_NS = _info.num_subcores
_L = _info.num_lanes
_NW = _NC * _NS
_BPW = _B // _NW
_IPW = _BPW * _F
_CH = 128
_NCH = _IPW // _CH
_OD = (_F + 1) * _D


def _body(idx_hbm, cont_hbm, table_hbm, wb_hbm, out_hbm,
          idx2, gidx, rows, slab, cont_v, wb_v, gsem, ssem):
  wid = lax.axis_index("s") * _NC + lax.axis_index("c")
  base_b = wid * _BPW

  pltpu.sync_copy(idx_hbm.at[pl.ds(base_b, _BPW)], idx2)
  for c in range(_C):
    pltpu.sync_copy(cont_hbm.at[pl.ds(c * _B + base_b, _BPW)],
                    cont_v.at[pl.ds(c * _BPW, _BPW)])
  pltpu.sync_copy(wb_hbm, wb_v)

  iota = lax.iota(jnp.int32, _L)

  fhi = iota + (_F - _L)
  vlo = iota * _VP
  vhi = fhi * _VP

  def gbody(b, _):
    bv = jnp.full((_L,), b, jnp.int32)
    lo = plsc.load_gather(idx2, [bv, iota])
    hi = plsc.load_gather(idx2, [bv, fhi])
    gidx[pl.ds(b * _F, _L)] = lo + vlo
    gidx[pl.ds(b * _F + (_F - _L), _L)] = hi + vhi
    return 0

  lax.fori_loop(0, _BPW, gbody, 0)

  gcopies = [
      pltpu.async_copy(table_hbm.at[gidx.at[pl.ds(j * _CH, _CH)]],
                       rows.at[pl.ds(j * _CH, _CH)], gsem)
      for j in range(_NCH)
  ]

  wsp = [[wb_v[c * _D + d] for d in range(_D)] for c in range(_C)]
  bsp = [wb_v[_C * _D + d] for d in range(_D)]

  def mbody(bc, _):
    bs = pl.ds(bc * _L, _L)
    xc = [cont_v[pl.ds(c * _BPW + bc * _L, _L)] for c in range(_C)]
    for d in range(_D):
      acc = bsp[d]
      for c in range(_C):
        acc = acc + xc[c] * wsp[c][d]
      slab[_F * _D + d, bs] = acc
    return 0

  lax.fori_loop(0, _BPW // _L, mbody, 0)

  for cp in gcopies:
    cp.wait()

  dcol = [jnp.full((_L,), d, jnp.int32) for d in range(_D)]

  def tbody(bc, _):
    bvec = bc * _L + iota
    bs = pl.ds(bc * _L, _L)
    for f in range(_F):
      rvec = bvec * _F + f
      for d in range(_D):
        slab[f * _D + d, bs] = plsc.load_gather(rows, [rvec, dcol[d]])
    return 0

  lax.fori_loop(0, _BPW // _L, tbody, 0)

  pltpu.async_copy(slab, out_hbm.at[:, pl.ds(base_b, _BPW)], ssem).wait()


_sc_call = pl.kernel(
    _body,
    out_type=jax.ShapeDtypeStruct((_OD, _B), jnp.float32),
    mesh=plsc.VectorSubcoreMesh(core_axis_name="c", subcore_axis_name="s"),
    compiler_params=pltpu.CompilerParams(use_tc_tiling_on_sc=False,
                                         needs_layout_passes=False),
    scratch_types=[
        pltpu.VMEM((_BPW, _FP), jnp.int32),
        pltpu.VMEM((_IPW,), jnp.int32),
        pltpu.VMEM((_IPW, _D), jnp.float32),
        pltpu.VMEM((_OD, _BPW), jnp.float32),
        pltpu.VMEM((_BPW * _C,), jnp.float32),
        pltpu.VMEM(((_C + 1) * _D, _L), jnp.float32),
        pltpu.SemaphoreType.DMA,
        pltpu.SemaphoreType.DMA,
    ],
)


def _repack_body(t_ref, o_ref):
  x = t_ref[0]
  x3 = x.T.reshape(_VB // 8, 8, _D)
  o_ref[...] = jnp.concatenate([x3[:, i, :] for i in range(8)], axis=1)


_repack = pl.pallas_call(
    _repack_body,
    grid=(_F, _NVB),
    in_specs=[pl.BlockSpec((1, _D, _VB), lambda f, vb: (f, 0, vb))],
    out_specs=pl.BlockSpec((_VB // 8, 8 * _D),
                           lambda f, vb: (f * _NVB + vb, 0)),
    out_shape=jax.ShapeDtypeStruct((_RP // 8, 8 * _D), jnp.float32),
    compiler_params=pltpu.CompilerParams(
        dimension_semantics=("parallel", "parallel")),
)


def kernel(indices, cont, tables, means, variances, W, b):
  scale = 1.0 / jnp.sqrt(variances)
  w_fold = W * scale[:, None]
  b_fold = b - (means * scale) @ W
  wb_flat = lax.optimization_barrier(
      jnp.tile(jnp.concatenate([w_fold.reshape(-1), b_fold])[:, None],
               (1, _L)))
  idx_pad = lax.optimization_barrier(
      jnp.pad(indices.astype(jnp.int32), ((0, 0), (0, _FP - _F))))
  cont_flat = lax.optimization_barrier(cont.T.reshape(-1))
  table_flat = _repack(jnp.transpose(tables, (0, 2, 1))).reshape(_RP, _D)
  out_t = _sc_call(idx_pad, cont_flat, table_flat, wb_flat)
  return jnp.transpose(out_t.reshape(_F + 1, _D, _B), (2, 0, 1))


# speedup vs baseline: 4.3853x; 1.1000x over previous
import jax
import jax.numpy as jnp
from jax import lax
from jax.experimental import pallas as pl
from jax.experimental.pallas import tpu as pltpu
from jax.experimental.pallas import tpu_sc as plsc

_B = 4096
_F = 26
_V = 100001
_D = 16
_C = 4
_FP = 32
_VP = 100096
_RP = _F * _VP
_VB = 4352
_NVB = _VP // _VB

_info = plsc.get_sparse_core_info()
_NC = _info.num_cores
_NS = _info.num_subcores
_L = _info.num_lanes
_NW = _NC * _NS
_BPW = _B // _NW
_IPW = _BPW * _F
_CH = 128
_NCH = _IPW // _CH
_OD = (_F + 1) * _D


def _body(idx_hbm, cont_hbm, table_hbm, wb_hbm, out_hbm,
          idx2, gidx, rows, slab, cont_v, wb_v, gsem, ssem):
  wid = lax.axis_index("s") * _NC + lax.axis_index("c")
  base_b = wid * _BPW

  pltpu.sync_copy(idx_hbm.at[pl.ds(base_b, _BPW)], idx2)
  for c in range(_C):
    pltpu.sync_copy(cont_hbm.at[pl.ds(c * _B + base_b, _BPW)],
                    cont_v.at[pl.ds(c * _BPW, _BPW)])
  pltpu.sync_copy(wb_hbm, wb_v)

  iota = lax.iota(jnp.int32, _L)

  fhi = iota + (_F - _L)
  vlo = iota * _VP
  vhi = fhi * _VP

  def gbody(b, _):
    bv = jnp.full((_L,), b, jnp.int32)
    lo = plsc.load_gather(idx2, [bv, iota])
    hi = plsc.load_gather(idx2, [bv, fhi])
    gidx[pl.ds(b * _F, _L)] = lo + vlo
    gidx[pl.ds(b * _F + (_F - _L), _L)] = hi + vhi
    return 0

  lax.fori_loop(0, _BPW, gbody, 0)

  gcopies = [
      pltpu.async_copy(table_hbm.at[gidx.at[pl.ds(j * _CH, _CH)]],
                       rows.at[pl.ds(j * _CH, _CH)], gsem)
      for j in range(_NCH)
  ]

  wsp = [[wb_v[c * _D + d] for d in range(_D)] for c in range(_C)]
  bsp = [wb_v[_C * _D + d] for d in range(_D)]

  def mbody(bc, _):
    bs = pl.ds(bc * _L, _L)
    xc = [cont_v[pl.ds(c * _BPW + bc * _L, _L)] for c in range(_C)]
    for d in range(_D):
      acc = bsp[d]
      for c in range(_C):
        acc = acc + xc[c] * wsp[c][d]
      slab[_F * _D + d, bs] = acc
    return 0

  lax.fori_loop(0, _BPW // _L, mbody, 0)

  for cp in gcopies:
    cp.wait()

  dcol = [jnp.full((_L,), d, jnp.int32) for d in range(_D)]

  def tbody(bc, _):
    bvec = bc * _L + iota
    bs = pl.ds(bc * _L, _L)
    for f in range(_F):
      rvec = bvec * _F + f
      for d in range(_D):
        slab[f * _D + d, bs] = plsc.load_gather(rows, [rvec, dcol[d]])
    return 0

  lax.fori_loop(0, _BPW // _L, tbody, 0)

  pltpu.async_copy(slab, out_hbm.at[:, pl.ds(base_b, _BPW)], ssem).wait()


_sc_call = pl.kernel(
    _body,
    out_type=jax.ShapeDtypeStruct((_OD, _B), jnp.float32),
    mesh=plsc.VectorSubcoreMesh(core_axis_name="c", subcore_axis_name="s"),
    compiler_params=pltpu.CompilerParams(use_tc_tiling_on_sc=False,
                                         needs_layout_passes=False),
    scratch_types=[
        pltpu.VMEM((_BPW, _FP), jnp.int32),
        pltpu.VMEM((_IPW,), jnp.int32),
        pltpu.VMEM((_IPW, _D), jnp.float32),
        pltpu.VMEM((_OD, _BPW), jnp.float32),
        pltpu.VMEM((_BPW * _C,), jnp.float32),
        pltpu.VMEM(((_C + 1) * _D, _L), jnp.float32),
        pltpu.SemaphoreType.DMA,
        pltpu.SemaphoreType.DMA,
    ],
)


def _repack_body(t_ref, o_ref):
  x = t_ref[0]
  x3 = x.T.reshape(_VB // 8, 8, _D)
  y = jnp.stack([x3[:, i, :] for i in range(8)], axis=1)
  o_ref[...] = y.reshape(_VB // 8, 8 * _D)


_repack = pl.pallas_call(
    _repack_body,
    grid=(_F, _NVB),
    in_specs=[pl.BlockSpec((1, _D, _VB), lambda f, vb: (f, 0, vb))],
    out_specs=pl.BlockSpec((_VB // 8, 8 * _D),
                           lambda f, vb: (f * _NVB + vb, 0)),
    out_shape=jax.ShapeDtypeStruct((_RP // 8, 8 * _D), jnp.float32),
    compiler_params=pltpu.CompilerParams(
        dimension_semantics=("parallel", "parallel")),
)


def kernel(indices, cont, tables, means, variances, W, b):
  scale = 1.0 / jnp.sqrt(variances)
  w_fold = W * scale[:, None]
  b_fold = b - (means * scale) @ W
  wb_flat = lax.optimization_barrier(
      jnp.tile(jnp.concatenate([w_fold.reshape(-1), b_fold])[:, None],
               (1, _L)))
  idx_pad = lax.optimization_barrier(
      jnp.pad(indices.astype(jnp.int32), ((0, 0), (0, _FP - _F))))
  cont_flat = lax.optimization_barrier(cont.T.reshape(-1))
  table_flat = _repack(jnp.transpose(tables, (0, 2, 1))).reshape(_RP, _D)
  out_t = _sc_call(idx_pad, cont_flat, table_flat, wb_flat)
  return jnp.transpose(out_t.reshape(_F + 1, _D, _B), (2, 0, 1))
